# baseline (speedup 1.0000x reference)
import functools

import numpy as np
import jax
import jax.numpy as jnp
from jax import lax
from jax.experimental import pallas as pl
from jax.experimental.pallas import tpu as pltpu
from jax.experimental.pallas import tpu_sc as plsc

HIDDEN = 64
ROWW = 128
HALF = HIDDEN // 2

_j = np.arange(HIDDEN)
_ang = 2.0 * np.pi * np.outer(_j, _j) / HIDDEN
_WR = np.cos(_ang).astype(np.float32)
_WI = (-np.sin(_ang)).astype(np.float32)


@functools.cache
def _make_sc_gather(B: int):
    info = plsc.get_sparse_core_info()
    NC, NS = info.num_cores, info.num_subcores
    NW = NC * NS
    assert B % NW == 0
    bpw = B // NW
    CH = 128
    assert bpw % CH == 0
    nch = bpw // CH
    mesh = plsc.VectorSubcoreMesh(core_axis_name="c", subcore_axis_name="s")

    @functools.partial(
        pl.kernel,
        mesh=mesh,
        out_type=(
            jax.ShapeDtypeStruct((B, ROWW), jnp.float32),
            jax.ShapeDtypeStruct((B, ROWW), jnp.float32),
            jax.ShapeDtypeStruct((B, ROWW), jnp.float32),
        ),
        scratch_types=[
            pltpu.VMEM((bpw,), jnp.int32),
            pltpu.VMEM((bpw,), jnp.int32),
            pltpu.VMEM((bpw,), jnp.int32),
            pltpu.VMEM((2, CH, ROWW), jnp.float32),
            pltpu.VMEM((2, CH, ROWW), jnp.float32),
            pltpu.VMEM((2, CH, ROWW), jnp.float32),
            pltpu.SemaphoreType.DMA,
            pltpu.SemaphoreType.DMA,
            pltpu.SemaphoreType.DMA,
        ],
    )
    def sc_gather(h_hbm, t_hbm, r_hbm, ent2_hbm, rel2_hbm,
                  oh, ot, orel, hi_v, ti_v, ri_v, hbuf, tbuf, rbuf,
                  gsem0, gsem1, wsem):
        wid = lax.axis_index("s") * NC + lax.axis_index("c")
        base = wid * bpw
        pltpu.sync_copy(h_hbm.at[pl.ds(base, bpw)], hi_v)
        pltpu.sync_copy(t_hbm.at[pl.ds(base, bpw)], ti_v)
        pltpu.sync_copy(r_hbm.at[pl.ds(base, bpw)], ri_v)
        gsems = (gsem0, gsem1)

        def fire(c, slot):
            sl = pl.ds(c * CH, CH)
            sem = gsems[slot]
            pltpu.async_copy(ent2_hbm.at[hi_v.at[sl]], hbuf.at[slot], sem)
            pltpu.async_copy(ent2_hbm.at[ti_v.at[sl]], tbuf.at[slot], sem)
            pltpu.async_copy(rel2_hbm.at[ri_v.at[sl]], rbuf.at[slot], sem)

        def drain_gather(slot):
            sem = gsems[slot]
            pltpu.make_async_copy(
                ent2_hbm.at[pl.ds(0, CH)], hbuf.at[slot], sem).wait()
            pltpu.make_async_copy(
                ent2_hbm.at[pl.ds(0, CH)], tbuf.at[slot], sem).wait()
            pltpu.make_async_copy(
                rel2_hbm.at[pl.ds(0, CH)], rbuf.at[slot], sem).wait()

        def write(c, slot):
            osl = pl.ds(base + c * CH, CH)
            pltpu.async_copy(hbuf.at[slot], oh.at[osl], wsem)
            pltpu.async_copy(tbuf.at[slot], ot.at[osl], wsem)
            pltpu.async_copy(rbuf.at[slot], orel.at[osl], wsem)

        def drain_write(slot):
            pltpu.make_async_copy(
                ent2_hbm.at[pl.ds(0, CH)], hbuf.at[slot], wsem).wait()
            pltpu.make_async_copy(
                ent2_hbm.at[pl.ds(0, CH)], tbuf.at[slot], wsem).wait()
            pltpu.make_async_copy(
                rel2_hbm.at[pl.ds(0, CH)], rbuf.at[slot], wsem).wait()

        fire(0, 0)
        for c in range(nch):
            slot = c % 2
            nxt = 1 - slot
            if c + 1 < nch:
                if c >= 1:
                    drain_write(nxt)
                fire(c + 1, nxt)
            drain_gather(slot)
            write(c, slot)
        drain_write(0)
        drain_write(1)

    return sc_gather


def _bf16_pack(x):
    u = lax.bitcast_convert_type(x, jnp.uint32)
    r = (u + jnp.uint32(0x7FFF) + ((u >> 16) & jnp.uint32(1))) >> 16
    packed = r[:HALF, :] | (r[HALF:, :] << 16)
    return lax.bitcast_convert_type(packed, jnp.float32)


def _repack_body(x0_ref, x1_ref, x2_ref, x3_ref, out_ref):
    qs = []
    for ref in (x0_ref, x1_ref, x2_ref, x3_ref):
        qs.append(_bf16_pack(ref[...]).T)
    out_ref[...] = jnp.concatenate(qs, axis=1)


def _repack(tableT, blk, nblk):
    n = tableT.shape[1]
    lastb = pl.cdiv(n, blk) - 1

    def mk(q):
        return pl.BlockSpec(
            (HIDDEN, blk), lambda i, q=q: (0, jnp.minimum(nblk * q + i, lastb)))

    return pl.pallas_call(
        _repack_body,
        grid=(nblk,),
        in_specs=[mk(0), mk(1), mk(2), mk(3)],
        out_specs=pl.BlockSpec((blk, ROWW), lambda i: (i, 0)),
        out_shape=jax.ShapeDtypeStruct((nblk * blk, ROWW), jnp.float32),
    )(tableT, tableT, tableT, tableT)


def _bf16_unpack(x32):
    u = lax.bitcast_convert_type(x32, jnp.uint32)
    lo = lax.bitcast_convert_type(u << 16, jnp.float32)
    hi = lax.bitcast_convert_type(u & jnp.uint32(0xFFFF0000), jnp.float32)
    return jnp.concatenate([lo, hi], axis=1)


def _quarter_select(x, q):
    s01 = jnp.where(q < 1, x[:, 0 * HALF:1 * HALF], x[:, 1 * HALF:2 * HALF])
    s23 = jnp.where(q < 3, x[:, 2 * HALF:3 * HALF], x[:, 3 * HALF:4 * HALF])
    return jnp.where(q < 2, s01, s23)


def _tc_body(h_ref, t_ref, r_ref, qh_ref, qt_ref, qr_ref, wr_ref, wi_ref,
             out_ref):
    f32 = jnp.float32
    h = _bf16_unpack(_quarter_select(h_ref[...], qh_ref[...]))
    t = _bf16_unpack(_quarter_select(t_ref[...], qt_ref[...]))
    r = _bf16_unpack(_quarter_select(r_ref[...], qr_ref[...]))
    wr = wr_ref[...]
    wi = wi_ref[...]
    hr = jnp.dot(h, wr, preferred_element_type=f32)
    hi = jnp.dot(h, wi, preferred_element_type=f32)
    tr = jnp.dot(t, wr, preferred_element_type=f32)
    ti = jnp.dot(t, wi, preferred_element_type=f32)
    rr = jnp.dot(r, wr, preferred_element_type=f32)
    ri = jnp.dot(r, wi, preferred_element_type=f32)
    p = (hr * tr + hi * ti) * rr + (hr * ti - hi * tr) * ri
    s = jnp.sum(p, axis=1, keepdims=True) * (1.0 / HIDDEN)
    nrm = lax.rsqrt(jnp.maximum(jnp.sum(r * r, axis=1, keepdims=True), 1e-12))
    out_ref[...] = -jax.nn.sigmoid(s * nrm)


def _tc_score(h4_e, t4_e, r4_e, qh, qt, qr, interpret=False):
    B = h4_e.shape[0]
    BLK = min(B, 2048)
    assert B % BLK == 0
    wr = jnp.asarray(_WR)
    wi = jnp.asarray(_WI)
    return pl.pallas_call(
        _tc_body,
        grid=(B // BLK,),
        in_specs=[
            pl.BlockSpec((BLK, ROWW), lambda i: (i, 0)),
            pl.BlockSpec((BLK, ROWW), lambda i: (i, 0)),
            pl.BlockSpec((BLK, ROWW), lambda i: (i, 0)),
            pl.BlockSpec((BLK, 1), lambda i: (i, 0)),
            pl.BlockSpec((BLK, 1), lambda i: (i, 0)),
            pl.BlockSpec((BLK, 1), lambda i: (i, 0)),
            pl.BlockSpec((HIDDEN, HIDDEN), lambda i: (0, 0)),
            pl.BlockSpec((HIDDEN, HIDDEN), lambda i: (0, 0)),
        ],
        out_specs=pl.BlockSpec((BLK, 1), lambda i: (i, 0)),
        out_shape=jax.ShapeDtypeStruct((B, 1), jnp.float32),
        interpret=interpret,
    )(h4_e, t4_e, r4_e, qh, qt, qr, wr, wi)


def kernel(h, t, r, ent_embeddings, rel_embeddings):
    h = h.astype(jnp.int32)
    t = t.astype(jnp.int32)
    r = r.astype(jnp.int32)
    B = h.shape[0]
    BLK_E, NBLK_E = 2048, 123
    BLK_R, NBLK_R = 256, 1
    S_E = BLK_E * NBLK_E
    S_R = BLK_R * NBLK_R
    assert 4 * S_E >= ent_embeddings.shape[0]
    assert 4 * S_R >= rel_embeddings.shape[0]
    ent4 = _repack(ent_embeddings.T, BLK_E, NBLK_E)
    rel4 = _repack(rel_embeddings.T, BLK_R, NBLK_R)
    qh = (h // S_E).astype(jnp.int32).reshape(B, 1)
    qt = (t // S_E).astype(jnp.int32).reshape(B, 1)
    qr = (r // S_R).astype(jnp.int32).reshape(B, 1)
    gather = _make_sc_gather(B)
    h4_e, t4_e, r4_e = gather(h % S_E, t % S_E, r % S_R, ent4, rel4)
    return _tc_score(h4_e, t4_e, r4_e, qh, qt, qr)


# speedup vs baseline: 1.0448x; 1.0448x over previous
import functools

import numpy as np
import jax
import jax.numpy as jnp
from jax import lax
from jax.experimental import pallas as pl
from jax.experimental.pallas import tpu as pltpu
from jax.experimental.pallas import tpu_sc as plsc

HIDDEN = 64
ROWW = 128
HALF = HIDDEN // 2

_j = np.arange(HIDDEN)
_ang = 2.0 * np.pi * np.outer(_j, _j) / HIDDEN
_WR = np.cos(_ang).astype(np.float32)
_WI = (-np.sin(_ang)).astype(np.float32)


@functools.cache
def _make_sc_gather(B: int):
    info = plsc.get_sparse_core_info()
    NC, NS = info.num_cores, info.num_subcores
    NW = NC * NS
    assert B % NW == 0
    bpw = B // NW
    CH = 128
    assert bpw % CH == 0
    nch = bpw // CH
    mesh = plsc.VectorSubcoreMesh(core_axis_name="c", subcore_axis_name="s")

    @functools.partial(
        pl.kernel,
        mesh=mesh,
        out_type=(
            jax.ShapeDtypeStruct((B, ROWW), jnp.float32),
            jax.ShapeDtypeStruct((B, ROWW), jnp.float32),
            jax.ShapeDtypeStruct((B, ROWW), jnp.float32),
        ),
        scratch_types=[
            pltpu.VMEM((bpw,), jnp.int32),
            pltpu.VMEM((bpw,), jnp.int32),
            pltpu.VMEM((bpw,), jnp.int32),
            pltpu.VMEM((2, CH, ROWW), jnp.float32),
            pltpu.VMEM((2, CH, ROWW), jnp.float32),
            pltpu.VMEM((2, CH, ROWW), jnp.float32),
            pltpu.SemaphoreType.DMA,
            pltpu.SemaphoreType.DMA,
            pltpu.SemaphoreType.DMA,
        ],
    )
    def sc_gather(h_hbm, t_hbm, r_hbm, ent2_hbm, rel2_hbm,
                  oh, ot, orel, hi_v, ti_v, ri_v, hbuf, tbuf, rbuf,
                  gsem0, gsem1, wsem):
        wid = lax.axis_index("s") * NC + lax.axis_index("c")
        base = wid * bpw
        pltpu.sync_copy(h_hbm.at[pl.ds(base, bpw)], hi_v)
        pltpu.sync_copy(t_hbm.at[pl.ds(base, bpw)], ti_v)
        pltpu.sync_copy(r_hbm.at[pl.ds(base, bpw)], ri_v)
        gsems = (gsem0, gsem1)

        def fire(c, slot):
            sl = pl.ds(c * CH, CH)
            sem = gsems[slot]
            pltpu.async_copy(ent2_hbm.at[hi_v.at[sl]], hbuf.at[slot], sem)
            pltpu.async_copy(ent2_hbm.at[ti_v.at[sl]], tbuf.at[slot], sem)
            pltpu.async_copy(rel2_hbm.at[ri_v.at[sl]], rbuf.at[slot], sem)

        def drain_gather(slot):
            sem = gsems[slot]
            pltpu.make_async_copy(
                ent2_hbm.at[pl.ds(0, CH)], hbuf.at[slot], sem).wait()
            pltpu.make_async_copy(
                ent2_hbm.at[pl.ds(0, CH)], tbuf.at[slot], sem).wait()
            pltpu.make_async_copy(
                rel2_hbm.at[pl.ds(0, CH)], rbuf.at[slot], sem).wait()

        def write(c, slot):
            osl = pl.ds(base + c * CH, CH)
            pltpu.async_copy(hbuf.at[slot], oh.at[osl], wsem)
            pltpu.async_copy(tbuf.at[slot], ot.at[osl], wsem)
            pltpu.async_copy(rbuf.at[slot], orel.at[osl], wsem)

        def drain_write(slot):
            pltpu.make_async_copy(
                ent2_hbm.at[pl.ds(0, CH)], hbuf.at[slot], wsem).wait()
            pltpu.make_async_copy(
                ent2_hbm.at[pl.ds(0, CH)], tbuf.at[slot], wsem).wait()
            pltpu.make_async_copy(
                rel2_hbm.at[pl.ds(0, CH)], rbuf.at[slot], wsem).wait()

        fire(0, 0)
        for c in range(nch):
            slot = c % 2
            nxt = 1 - slot
            if c + 1 < nch:
                if c >= 1:
                    drain_write(nxt)
                fire(c + 1, nxt)
            drain_gather(slot)
            write(c, slot)
        drain_write(0)
        drain_write(1)

    return sc_gather


def _bf16_pack(x):
    u = lax.bitcast_convert_type(x, jnp.uint32)
    r = (u + jnp.uint32(0x7FFF) + ((u >> 16) & jnp.uint32(1))) >> 16
    packed = r[:HALF, :] | (r[HALF:, :] << 16)
    return lax.bitcast_convert_type(packed, jnp.float32)


def _repack_body(x0_ref, x1_ref, x2_ref, x3_ref, out_ref):
    qs = []
    for ref in (x0_ref, x1_ref, x2_ref, x3_ref):
        qs.append(_bf16_pack(ref[...]).T)
    out_ref[...] = jnp.concatenate(qs, axis=1)


def _repack(tableT, blk, nblk):
    n = tableT.shape[1]
    lastb = pl.cdiv(n, blk) - 1

    def mk(q):
        return pl.BlockSpec(
            (HIDDEN, blk), lambda i, q=q: (0, jnp.minimum(nblk * q + i, lastb)))

    return pl.pallas_call(
        _repack_body,
        grid=(nblk,),
        in_specs=[mk(0), mk(1), mk(2), mk(3)],
        out_specs=pl.BlockSpec((blk, ROWW), lambda i: (i, 0)),
        out_shape=jax.ShapeDtypeStruct((nblk * blk, ROWW), jnp.float32),
    )(tableT, tableT, tableT, tableT)


def _bf16_unpack(x32):
    u = lax.bitcast_convert_type(x32, jnp.uint32)
    lo = lax.bitcast_convert_type(u << 16, jnp.float32)
    hi = lax.bitcast_convert_type(u & jnp.uint32(0xFFFF0000), jnp.float32)
    return jnp.concatenate([lo, hi], axis=1)


def _quarter_select(x, q):
    s01 = jnp.where(q < 1, x[:, 0 * HALF:1 * HALF], x[:, 1 * HALF:2 * HALF])
    s23 = jnp.where(q < 3, x[:, 2 * HALF:3 * HALF], x[:, 3 * HALF:4 * HALF])
    return jnp.where(q < 2, s01, s23)


def _tc_body(h_ref, t_ref, r_ref, qh_ref, qt_ref, qr_ref, wr_ref, wi_ref,
             out_ref):
    f32 = jnp.float32
    h = _bf16_unpack(_quarter_select(h_ref[...], qh_ref[...]))
    t = _bf16_unpack(_quarter_select(t_ref[...], qt_ref[...]))
    r = _bf16_unpack(_quarter_select(r_ref[...], qr_ref[...]))
    wr = wr_ref[...]
    wi = wi_ref[...]
    hr = jnp.dot(h, wr, preferred_element_type=f32)
    hi = jnp.dot(h, wi, preferred_element_type=f32)
    tr = jnp.dot(t, wr, preferred_element_type=f32)
    ti = jnp.dot(t, wi, preferred_element_type=f32)
    rr = jnp.dot(r, wr, preferred_element_type=f32)
    ri = jnp.dot(r, wi, preferred_element_type=f32)
    p = (hr * tr + hi * ti) * rr + (hr * ti - hi * tr) * ri
    s = jnp.sum(p, axis=1, keepdims=True) * (1.0 / HIDDEN)
    nrm = lax.rsqrt(jnp.maximum(jnp.sum(r * r, axis=1, keepdims=True), 1e-12))
    out_ref[...] = -jax.nn.sigmoid(s * nrm)


def _tc_score(h4_e, t4_e, r4_e, qh, qt, qr, interpret=False):
    B = h4_e.shape[0]
    BLK = min(B, 2048)
    assert B % BLK == 0
    wr = jnp.asarray(_WR)
    wi = jnp.asarray(_WI)
    return pl.pallas_call(
        _tc_body,
        grid=(B // BLK,),
        in_specs=[
            pl.BlockSpec((BLK, ROWW), lambda i: (i, 0)),
            pl.BlockSpec((BLK, ROWW), lambda i: (i, 0)),
            pl.BlockSpec((BLK, ROWW), lambda i: (i, 0)),
            pl.BlockSpec((BLK, 1), lambda i: (i, 0)),
            pl.BlockSpec((BLK, 1), lambda i: (i, 0)),
            pl.BlockSpec((BLK, 1), lambda i: (i, 0)),
            pl.BlockSpec((HIDDEN, HIDDEN), lambda i: (0, 0)),
            pl.BlockSpec((HIDDEN, HIDDEN), lambda i: (0, 0)),
        ],
        out_specs=pl.BlockSpec((BLK, 1), lambda i: (i, 0)),
        out_shape=jax.ShapeDtypeStruct((B, 1), jnp.float32),
        interpret=interpret,
    )(h4_e, t4_e, r4_e, qh, qt, qr, wr, wi)


def kernel(h, t, r, ent_embeddings, rel_embeddings):
    h = h.astype(jnp.int32)
    t = t.astype(jnp.int32)
    r = r.astype(jnp.int32)
    B = h.shape[0]
    BLK_E, NBLK_E = 8192, 31
    BLK_R, NBLK_R = 256, 1
    S_E = BLK_E * NBLK_E
    S_R = BLK_R * NBLK_R
    assert 4 * S_E >= ent_embeddings.shape[0]
    assert 4 * S_R >= rel_embeddings.shape[0]
    ent4 = _repack(ent_embeddings.T, BLK_E, NBLK_E)
    rel4 = _repack(rel_embeddings.T, BLK_R, NBLK_R)
    qh = (h // S_E).astype(jnp.int32).reshape(B, 1)
    qt = (t // S_E).astype(jnp.int32).reshape(B, 1)
    qr = (r // S_R).astype(jnp.int32).reshape(B, 1)
    gather = _make_sc_gather(B)
    h4_e, t4_e, r4_e = gather(h % S_E, t % S_E, r % S_R, ent4, rel4)
    return _tc_score(h4_e, t4_e, r4_e, qh, qt, qr)


# speedup vs baseline: 1.2609x; 1.2069x over previous
import functools

import numpy as np
import jax
import jax.numpy as jnp
from jax import lax
from jax.experimental import pallas as pl
from jax.experimental.pallas import tpu as pltpu
from jax.experimental.pallas import tpu_sc as plsc

HIDDEN = 64
ROWW = 128
HALF = HIDDEN // 2

_j = np.arange(HIDDEN)
_ang = 2.0 * np.pi * np.outer(_j, _j) / HIDDEN
_WR = np.cos(_ang).astype(np.float32)
_WI = (-np.sin(_ang)).astype(np.float32)


@functools.cache
def _make_sc_gather(B: int):
    info = plsc.get_sparse_core_info()
    NC, NS = info.num_cores, info.num_subcores
    NW = NC * NS
    assert B % NW == 0
    bpw = B // NW
    CH = 128
    assert bpw % CH == 0
    nch = bpw // CH
    mesh = plsc.VectorSubcoreMesh(core_axis_name="c", subcore_axis_name="s")

    @functools.partial(
        pl.kernel,
        mesh=mesh,
        out_type=(
            jax.ShapeDtypeStruct((B, ROWW), jnp.float32),
            jax.ShapeDtypeStruct((B, ROWW), jnp.float32),
            jax.ShapeDtypeStruct((B, ROWW), jnp.float32),
        ),
        scratch_types=[
            pltpu.VMEM((bpw,), jnp.int32),
            pltpu.VMEM((bpw,), jnp.int32),
            pltpu.VMEM((bpw,), jnp.int32),
            pltpu.VMEM((2, CH, ROWW), jnp.float32),
            pltpu.VMEM((2, CH, ROWW), jnp.float32),
            pltpu.VMEM((2, CH, ROWW), jnp.float32),
            pltpu.SemaphoreType.DMA,
            pltpu.SemaphoreType.DMA,
            pltpu.SemaphoreType.DMA,
        ],
    )
    def sc_gather(h_hbm, t_hbm, r_hbm, ent2_hbm, rel2_hbm,
                  oh, ot, orel, hi_v, ti_v, ri_v, hbuf, tbuf, rbuf,
                  gsem0, gsem1, wsem):
        wid = lax.axis_index("s") * NC + lax.axis_index("c")
        base = wid * bpw
        pltpu.sync_copy(h_hbm.at[pl.ds(base, bpw)], hi_v)
        pltpu.sync_copy(t_hbm.at[pl.ds(base, bpw)], ti_v)
        pltpu.sync_copy(r_hbm.at[pl.ds(base, bpw)], ri_v)
        gsems = (gsem0, gsem1)

        def fire(c, slot):
            sl = pl.ds(c * CH, CH)
            sem = gsems[slot]
            pltpu.async_copy(ent2_hbm.at[hi_v.at[sl]], hbuf.at[slot], sem)
            pltpu.async_copy(ent2_hbm.at[ti_v.at[sl]], tbuf.at[slot], sem)
            pltpu.async_copy(rel2_hbm.at[ri_v.at[sl]], rbuf.at[slot], sem)

        def drain_gather(slot):
            sem = gsems[slot]
            pltpu.make_async_copy(
                ent2_hbm.at[pl.ds(0, CH)], hbuf.at[slot], sem).wait()
            pltpu.make_async_copy(
                ent2_hbm.at[pl.ds(0, CH)], tbuf.at[slot], sem).wait()
            pltpu.make_async_copy(
                rel2_hbm.at[pl.ds(0, CH)], rbuf.at[slot], sem).wait()

        def write(c, slot):
            osl = pl.ds(base + c * CH, CH)
            pltpu.async_copy(hbuf.at[slot], oh.at[osl], wsem)
            pltpu.async_copy(tbuf.at[slot], ot.at[osl], wsem)
            pltpu.async_copy(rbuf.at[slot], orel.at[osl], wsem)

        def drain_write(slot):
            pltpu.make_async_copy(
                ent2_hbm.at[pl.ds(0, CH)], hbuf.at[slot], wsem).wait()
            pltpu.make_async_copy(
                ent2_hbm.at[pl.ds(0, CH)], tbuf.at[slot], wsem).wait()
            pltpu.make_async_copy(
                rel2_hbm.at[pl.ds(0, CH)], rbuf.at[slot], wsem).wait()

        fire(0, 0)
        for c in range(nch):
            slot = c % 2
            nxt = 1 - slot
            if c + 1 < nch:
                if c >= 1:
                    drain_write(nxt)
                fire(c + 1, nxt)
            drain_gather(slot)
            write(c, slot)
        drain_write(0)
        drain_write(1)

    return sc_gather


def _repack_body(x1_ref, x2_ref, out_ref):
    x1 = x1_ref[...]
    x2 = x2_ref[...]
    out_ref[...] = jnp.concatenate([x1.T, x2.T], axis=1)


def _repack(tableT, blk, nblk):
    n = tableT.shape[1]
    lastb = pl.cdiv(n, blk) - 1

    def mk(q):
        return pl.BlockSpec(
            (HIDDEN, blk), lambda i, q=q: (0, jnp.minimum(nblk * q + i, lastb)))

    return pl.pallas_call(
        _repack_body,
        grid=(nblk,),
        in_specs=[mk(0), mk(1)],
        out_specs=pl.BlockSpec((blk, ROWW), lambda i: (i, 0)),
        out_shape=jax.ShapeDtypeStruct((nblk * blk, ROWW), jnp.float32),
    )(tableT, tableT)


def _half_select(x, q):
    return jnp.where(q < 1, x[:, :HIDDEN], x[:, HIDDEN:])


def _tc_body(h_ref, t_ref, r_ref, qh_ref, qt_ref, qr_ref, wr_ref, wi_ref,
             out_ref):
    f32 = jnp.float32
    h = _half_select(h_ref[...], qh_ref[...])
    t = _half_select(t_ref[...], qt_ref[...])
    r = _half_select(r_ref[...], qr_ref[...])
    wr = wr_ref[...]
    wi = wi_ref[...]
    hr = jnp.dot(h, wr, preferred_element_type=f32)
    hi = jnp.dot(h, wi, preferred_element_type=f32)
    tr = jnp.dot(t, wr, preferred_element_type=f32)
    ti = jnp.dot(t, wi, preferred_element_type=f32)
    rr = jnp.dot(r, wr, preferred_element_type=f32)
    ri = jnp.dot(r, wi, preferred_element_type=f32)
    p = (hr * tr + hi * ti) * rr + (hr * ti - hi * tr) * ri
    s = jnp.sum(p, axis=1, keepdims=True) * (1.0 / HIDDEN)
    nrm = lax.rsqrt(jnp.maximum(jnp.sum(r * r, axis=1, keepdims=True), 1e-12))
    out_ref[...] = -jax.nn.sigmoid(s * nrm)


def _tc_score(h4_e, t4_e, r4_e, qh, qt, qr, interpret=False):
    B = h4_e.shape[0]
    BLK = min(B, 2048)
    assert B % BLK == 0
    wr = jnp.asarray(_WR)
    wi = jnp.asarray(_WI)
    return pl.pallas_call(
        _tc_body,
        grid=(B // BLK,),
        in_specs=[
            pl.BlockSpec((BLK, ROWW), lambda i: (i, 0)),
            pl.BlockSpec((BLK, ROWW), lambda i: (i, 0)),
            pl.BlockSpec((BLK, ROWW), lambda i: (i, 0)),
            pl.BlockSpec((BLK, 1), lambda i: (i, 0)),
            pl.BlockSpec((BLK, 1), lambda i: (i, 0)),
            pl.BlockSpec((BLK, 1), lambda i: (i, 0)),
            pl.BlockSpec((HIDDEN, HIDDEN), lambda i: (0, 0)),
            pl.BlockSpec((HIDDEN, HIDDEN), lambda i: (0, 0)),
        ],
        out_specs=pl.BlockSpec((BLK, 1), lambda i: (i, 0)),
        out_shape=jax.ShapeDtypeStruct((B, 1), jnp.float32),
        interpret=interpret,
    )(h4_e, t4_e, r4_e, qh, qt, qr, wr, wi)


def kernel(h, t, r, ent_embeddings, rel_embeddings):
    h = h.astype(jnp.int32)
    t = t.astype(jnp.int32)
    r = r.astype(jnp.int32)
    B = h.shape[0]
    BLK_E, NBLK_E = 16384, 31
    BLK_R, NBLK_R = 512, 1
    S_E = BLK_E * NBLK_E
    S_R = BLK_R * NBLK_R
    assert 2 * S_E >= ent_embeddings.shape[0]
    assert 2 * S_R >= rel_embeddings.shape[0]
    ent4 = _repack(ent_embeddings.T, BLK_E, NBLK_E)
    rel4 = _repack(rel_embeddings.T, BLK_R, NBLK_R)
    qh = (h // S_E).astype(jnp.int32).reshape(B, 1)
    qt = (t // S_E).astype(jnp.int32).reshape(B, 1)
    qr = (r // S_R).astype(jnp.int32).reshape(B, 1)
    gather = _make_sc_gather(B)
    h4_e, t4_e, r4_e = gather(h % S_E, t % S_E, r % S_R, ent4, rel4)
    return _tc_score(h4_e, t4_e, r4_e, qh, qt, qr)
